# baseline (speedup 1.0000x reference)
import functools

import jax
import jax.numpy as jnp
from jax import lax
from jax.experimental import pallas as pl
from jax.experimental.pallas import tpu as pltpu
from jax.experimental.pallas import tpu_sc as plsc


def _make_gather(B, T, D):
    mesh = plsc.VectorSubcoreMesh(core_axis_name="c", subcore_axis_name="s")

    @functools.partial(
        pl.kernel,
        mesh=mesh,
        out_type=jax.ShapeDtypeStruct((B, D), jnp.float32),
        scratch_types=[
            pltpu.VMEM((B,), jnp.int32),
            pltpu.VMEM((B, D), jnp.float32),
            pltpu.SemaphoreType.DMA,
        ],
    )
    def k(xf_hbm, len_hbm, out_hbm, idx_v, rows_v, sem):
        c = lax.axis_index("c")
        s = lax.axis_index("s")
        wid = s * 2 + c

        @pl.when(wid == 0)
        def _():
            pltpu.sync_copy(len_hbm, idx_v)
            row = lax.iota(jnp.int32, B) * T + idx_v[...] - 1
            idx_v[...] = row
            pltpu.async_copy(xf_hbm.at[idx_v], rows_v, sem).wait()
            pltpu.sync_copy(rows_v, out_hbm)

    return k


def kernel(x, lengths):
    B, T, D = x.shape
    xflat = x.reshape(B * T, D)
    return _make_gather(B, T, D)(xflat, lengths.astype(jnp.int32))


# speedup vs baseline: 1.0670x; 1.0670x over previous
import functools

import jax
import jax.numpy as jnp
from jax import lax
from jax.experimental import pallas as pl
from jax.experimental.pallas import tpu as pltpu
from jax.experimental.pallas import tpu_sc as plsc


def _make_gather(B, T, D):
    mesh = plsc.VectorSubcoreMesh(
        core_axis_name="c", subcore_axis_name="s", num_cores=1
    )

    @functools.partial(
        pl.kernel,
        mesh=mesh,
        out_type=jax.ShapeDtypeStruct((B, D), jnp.float32),
        scratch_types=[
            pltpu.VMEM((B,), jnp.int32),
            pltpu.VMEM((B, D), jnp.float32),
            pltpu.SemaphoreType.DMA,
        ],
    )
    def k(xf_hbm, idx_hbm, out_hbm, idx_v, rows_v, sem):
        s = lax.axis_index("s")

        @pl.when(s == 0)
        def _():
            pltpu.sync_copy(idx_hbm, idx_v)
            pltpu.async_copy(xf_hbm.at[idx_v], rows_v, sem).wait()
            pltpu.sync_copy(rows_v, out_hbm)

    return k


def kernel(x, lengths):
    B, T, D = x.shape
    xflat = x.reshape(B * T, D)
    idx = jnp.arange(B, dtype=jnp.int32) * T + lengths.astype(jnp.int32) - 1
    return _make_gather(B, T, D)(xflat, idx)


# speedup vs baseline: 1.0969x; 1.0280x over previous
import functools

import jax
import jax.numpy as jnp
from jax import lax
from jax.experimental import pallas as pl
from jax.experimental.pallas import tpu as pltpu
from jax.experimental.pallas import tpu_sc as plsc


def _make_gather(B, T, D):
    mesh = plsc.VectorSubcoreMesh(
        core_axis_name="c", subcore_axis_name="s", num_cores=1
    )

    @functools.partial(
        pl.kernel,
        mesh=mesh,
        out_type=jax.ShapeDtypeStruct((B, D), jnp.float32),
        scratch_types=[
            pltpu.VMEM((B, 1), jnp.int32),
            pltpu.VMEM((1, D), jnp.float32),
            pltpu.SemaphoreType.DMA,
        ],
    )
    def k(xf_hbm, idx_hbm, out_hbm, idx_v, row_v, sem):
        s = lax.axis_index("s")
        pltpu.sync_copy(idx_hbm, idx_v)
        pltpu.async_copy(xf_hbm.at[idx_v.at[s]], row_v, sem).wait()
        pltpu.sync_copy(row_v, out_hbm.at[pl.ds(s, 1)])

    return k


def kernel(x, lengths):
    B, T, D = x.shape
    xflat = x.reshape(B * T, D)
    idx = jnp.arange(B, dtype=jnp.int32) * T + lengths.astype(jnp.int32) - 1
    return _make_gather(B, T, D)(xflat, idx.reshape(B, 1))


# speedup vs baseline: 1.1107x; 1.0127x over previous
import functools

import jax
import jax.numpy as jnp
from jax import lax
from jax.experimental import pallas as pl
from jax.experimental.pallas import tpu as pltpu
from jax.experimental.pallas import tpu_sc as plsc


def _make_gather(B, T, D):
    mesh = plsc.ScalarSubcoreMesh(axis_name="c", num_cores=1)

    @functools.partial(
        pl.kernel,
        mesh=mesh,
        out_type=jax.ShapeDtypeStruct((B, D), jnp.float32),
        scratch_types=[
            pltpu.SMEM((B,), jnp.int32),
            pltpu.SemaphoreType.DMA,
        ],
    )
    def k(xf_hbm, idx_hbm, out_hbm, idx_s, sem):
        pltpu.sync_copy(idx_hbm, idx_s)
        cps = []
        for b in range(B):
            r = idx_s[b]
            cps.append(
                pltpu.async_copy(
                    xf_hbm.at[pl.ds(r, 1)], out_hbm.at[pl.ds(b, 1)], sem
                )
            )
        for cp in cps:
            cp.wait()

    return k


def kernel(x, lengths):
    B, T, D = x.shape
    xflat = x.reshape(B * T, D)
    idx = jnp.arange(B, dtype=jnp.int32) * T + lengths.astype(jnp.int32) - 1
    return _make_gather(B, T, D)(xflat, idx)
